# baseline (speedup 1.0000x reference)
import functools

import jax
import jax.numpy as jnp
from jax import lax
from jax.experimental import pallas as pl
from jax.experimental.pallas import tpu as pltpu
from jax.experimental.pallas import tpu_sc as plsc

N = 10000
E = 320000
D = 128
K_HOPS = 2

NC = 2
NS = 16
NW = NC * NS
EPW = E // NW
CH = 125
NCHUNK = EPW // CH
NHALF = NCHUNK // 2
RPT = 624
RTAIL = N - NS * RPT
DW = D

def _sc_prop_body(src_hbm, dst_hbm, t_hbm, zeros_hbm, out_hbm,
                  acc_sh, srcidx, dstidx, rows0, rows1, sem0, sem1,
                  ssem0, ssem1):
    c = lax.axis_index("c")
    s = lax.axis_index("s")
    pltpu.sync_copy(zeros_hbm.at[pl.ds(s * RPT, RPT)],
                    acc_sh.at[pl.ds(s * RPT, RPT)])

    @pl.when(s == NS - 1)
    def _():
        pltpu.sync_copy(zeros_hbm.at[pl.ds(NS * RPT, RTAIL)],
                        acc_sh.at[pl.ds(NS * RPT, RTAIL)])

    plsc.subcore_barrier()

    wid = c * NS + s
    for h in range(2):
        pltpu.sync_copy(src_hbm.at[wid].at[h], srcidx)
        pltpu.sync_copy(dst_hbm.at[wid].at[h], dstidx)
        pltpu.async_copy(t_hbm.at[srcidx.at[0]], rows0, sem0)
        pltpu.async_copy(t_hbm.at[srcidx.at[1]], rows1, sem1)

        def chunk(j, carry):
            i0 = 2 * j
            pltpu.make_async_copy(t_hbm.at[srcidx.at[i0]], rows0, sem0).wait()
            pltpu.async_copy(rows0, acc_sh.at[dstidx.at[i0]], ssem0, add=True)
            pltpu.make_async_copy(t_hbm.at[srcidx.at[i0 + 1]], rows1,
                                  sem1).wait()
            pltpu.async_copy(rows1, acc_sh.at[dstidx.at[i0 + 1]], ssem1,
                             add=True)

            @pl.when(j < NHALF // 2 - 1)
            def _():
                pltpu.make_async_copy(rows0, acc_sh.at[dstidx.at[i0]],
                                      ssem0).wait()
                pltpu.async_copy(t_hbm.at[srcidx.at[i0 + 2]], rows0, sem0)
                pltpu.make_async_copy(rows1, acc_sh.at[dstidx.at[i0 + 1]],
                                      ssem1).wait()
                pltpu.async_copy(t_hbm.at[srcidx.at[i0 + 3]], rows1, sem1)

            return carry

        lax.fori_loop(0, NHALF // 2, chunk, 0)
        pltpu.make_async_copy(rows0, acc_sh.at[dstidx.at[NHALF - 2]],
                              ssem0).wait()
        pltpu.make_async_copy(rows1, acc_sh.at[dstidx.at[NHALF - 1]],
                              ssem1).wait()
    plsc.subcore_barrier()
    pltpu.sync_copy(acc_sh.at[pl.ds(s * RPT, RPT)],
                    out_hbm.at[c].at[pl.ds(s * RPT, RPT)])

    @pl.when(s == NS - 1)
    def _():
        pltpu.sync_copy(acc_sh.at[pl.ds(NS * RPT, RTAIL)],
                        out_hbm.at[c].at[pl.ds(NS * RPT, RTAIL)])


@functools.cache
def _get_sc_prop():
    mesh = plsc.VectorSubcoreMesh(core_axis_name="c", subcore_axis_name="s")
    return pl.kernel(
        _sc_prop_body,
        out_type=jax.ShapeDtypeStruct((NC, N, D), jnp.float32),
        mesh=mesh,
        scratch_types=[
            pltpu.VMEM_SHARED((N, D), jnp.float32),
            pltpu.VMEM((NHALF, CH), jnp.int32),
            pltpu.VMEM((NHALF, CH), jnp.int32),
            pltpu.VMEM((CH, D), jnp.float32),
            pltpu.VMEM((CH, D), jnp.float32),
            pltpu.SemaphoreType.DMA,
            pltpu.SemaphoreType.DMA,
            pltpu.SemaphoreType.DMA,
            pltpu.SemaphoreType.DMA,
        ],
    )


def _sc_prop(src, dst, t, zeros):
    return _get_sc_prop()(src, dst, t, zeros)


def _sc_deg_body(dst_hbm, ones_hbm, zeros_hbm, out_hbm, acc_sh, dstidx, ones_v):
    c = lax.axis_index("c")
    s = lax.axis_index("s")
    pltpu.sync_copy(zeros_hbm.at[pl.ds(s * RPT, RPT)],
                    acc_sh.at[pl.ds(s * RPT, RPT)])

    @pl.when(s == NS - 1)
    def _():
        pltpu.sync_copy(zeros_hbm.at[pl.ds(NS * RPT, RTAIL)],
                        acc_sh.at[pl.ds(NS * RPT, RTAIL)])

    plsc.subcore_barrier()
    wid = c * NS + s
    pltpu.sync_copy(ones_hbm, ones_v)
    for h in range(2):
        pltpu.sync_copy(dst_hbm.at[wid].at[h], dstidx)

        def chunk(i, carry):
            pltpu.sync_copy(ones_v, acc_sh.at[dstidx.at[i]], add=True)
            return carry

        lax.fori_loop(0, NHALF, chunk, 0)
    plsc.subcore_barrier()
    pltpu.sync_copy(acc_sh.at[pl.ds(s * RPT, RPT)],
                    out_hbm.at[c].at[pl.ds(s * RPT, RPT)])

    @pl.when(s == NS - 1)
    def _():
        pltpu.sync_copy(acc_sh.at[pl.ds(NS * RPT, RTAIL)],
                        out_hbm.at[c].at[pl.ds(NS * RPT, RTAIL)])


@functools.cache
def _get_sc_deg():
    mesh = plsc.VectorSubcoreMesh(core_axis_name="c", subcore_axis_name="s")
    return pl.kernel(
        _sc_deg_body,
        out_type=jax.ShapeDtypeStruct((NC, N, DW), jnp.float32),
        mesh=mesh,
        scratch_types=[
            pltpu.VMEM_SHARED((N, DW), jnp.float32),
            pltpu.VMEM((NHALF, CH), jnp.int32),
            pltpu.VMEM((CH, DW), jnp.float32),
        ],
    )


_BLK = 1000
_NBLK = N // _BLK


def _prep_body(x_ref, degp_ref, t0_ref, normb_ref):
    deg = degp_ref[0, :, :1] + degp_ref[1, :, :1]
    norm = jnp.broadcast_to(lax.rsqrt(jnp.maximum(deg, 1.0)), (_BLK, D))
    normb_ref[...] = norm
    t0_ref[...] = norm * jnp.nan_to_num(x_ref[...])


def _tc_prep(x, degp):
    return pl.pallas_call(
        _prep_body,
        grid=(_NBLK,),
        in_specs=[
            pl.BlockSpec((_BLK, D), lambda i: (i, 0)),
            pl.BlockSpec((NC, _BLK, DW), lambda i: (0, i, 0)),
        ],
        out_specs=[
            pl.BlockSpec((_BLK, D), lambda i: (i, 0)),
            pl.BlockSpec((_BLK, D), lambda i: (i, 0)),
        ],
        out_shape=[
            jax.ShapeDtypeStruct((N, D), jnp.float32),
            jax.ShapeDtypeStruct((N, D), jnp.float32),
        ],
    )(x, degp)


def _mid_body(sp_ref, normb_ref, t_ref):
    nb = normb_ref[...]
    t_ref[...] = nb * nb * (sp_ref[0] + sp_ref[1])


def _tc_mid(sp, normb):
    return pl.pallas_call(
        _mid_body,
        grid=(_NBLK,),
        in_specs=[
            pl.BlockSpec((NC, _BLK, D), lambda i: (0, i, 0)),
            pl.BlockSpec((_BLK, D), lambda i: (i, 0)),
        ],
        out_specs=pl.BlockSpec((_BLK, D), lambda i: (i, 0)),
        out_shape=jax.ShapeDtypeStruct((N, D), jnp.float32),
    )(sp, normb)


def _layer_body(feat_ref, s1_ref, s2_ref, normb_ref, w_ref, b_ref,
                h_ref, t_ref):
    nb = normb_ref[...]
    f = jnp.nan_to_num(feat_ref[...])
    h1 = nb * (s1_ref[0] + s1_ref[1])
    h2 = nb * (s2_ref[0] + s2_ref[1])
    cat = jnp.concatenate([f, h1, h2], axis=1)
    out = jnp.dot(cat, w_ref[...], preferred_element_type=jnp.float32)
    out = jnp.maximum(out + b_ref[...], 0.0)
    h_ref[...] = out
    t_ref[...] = nb * out


def _tc_layer(feat, s1p, s2p, normb, W, b):
    return pl.pallas_call(
        _layer_body,
        grid=(_NBLK,),
        in_specs=[
            pl.BlockSpec((_BLK, D), lambda i: (i, 0)),
            pl.BlockSpec((NC, _BLK, D), lambda i: (0, i, 0)),
            pl.BlockSpec((NC, _BLK, D), lambda i: (0, i, 0)),
            pl.BlockSpec((_BLK, D), lambda i: (i, 0)),
            pl.BlockSpec((3 * D, D), lambda i: (0, 0)),
            pl.BlockSpec((1, D), lambda i: (0, 0)),
        ],
        out_specs=[
            pl.BlockSpec((_BLK, D), lambda i: (i, 0)),
            pl.BlockSpec((_BLK, D), lambda i: (i, 0)),
        ],
        out_shape=[
            jax.ShapeDtypeStruct((N, D), jnp.float32),
            jax.ShapeDtypeStruct((N, D), jnp.float32),
        ],
    )(feat, s1p, s2p, normb, W, b)


def _final_body(h_ref, wc_ref, bc_ref, out_ref, acc_ref):
    i = pl.program_id(0)

    @pl.when(i == 0)
    def _():
        acc_ref[...] = jnp.zeros_like(acc_ref)

    acc_ref[...] += jnp.sum(h_ref[...], axis=0, keepdims=True)

    @pl.when(i == _NBLK - 1)
    def _():
        hg = acc_ref[...] * (1.0 / N)
        out_ref[...] = jnp.dot(hg, wc_ref[...],
                               preferred_element_type=jnp.float32) + bc_ref[...]


def _tc_final(h, Wc, bc):
    return pl.pallas_call(
        _final_body,
        grid=(_NBLK,),
        in_specs=[
            pl.BlockSpec((_BLK, D), lambda i: (i, 0)),
            pl.BlockSpec((D, 1), lambda i: (0, 0)),
            pl.BlockSpec((1, 1), lambda i: (0, 0)),
        ],
        out_specs=pl.BlockSpec((1, 1), lambda i: (0, 0)),
        out_shape=jax.ShapeDtypeStruct((1, 1), jnp.float32),
        scratch_shapes=[pltpu.VMEM((1, D), jnp.float32)],
    )(h, Wc, bc)


def kernel(x, edge_index, W1, b1, W2, b2, W3, b3, Wc, bc):
    src = edge_index[0].reshape(NW, 2, NHALF, CH)
    dst = edge_index[1].reshape(NW, 2, NHALF, CH)
    zeros = jnp.zeros((N, D), jnp.float32)
    ones_chunk = jnp.ones((CH, DW), jnp.float32)
    b1r = b1.reshape(1, D)
    b2r = b2.reshape(1, D)
    b3r = b3.reshape(1, D)
    bcr = bc.reshape(1, 1)

    degp = _get_sc_deg()(dst, ones_chunk, zeros)
    t, normb = _tc_prep(x, degp)

    feat = x
    for W, b in ((W1, b1r), (W2, b2r), (W3, b3r)):
        s1p = _sc_prop(src, dst, t, zeros)
        t1 = _tc_mid(s1p, normb)
        s2p = _sc_prop(src, dst, t1, zeros)
        feat, t = _tc_layer(feat, s1p, s2p, normb, W, b)

    return _tc_final(feat, Wc, bcr)


# speedup vs baseline: 1.0939x; 1.0939x over previous
import functools

import jax
import jax.numpy as jnp
from jax import lax
from jax.experimental import pallas as pl
from jax.experimental.pallas import tpu as pltpu
from jax.experimental.pallas import tpu_sc as plsc

N = 10000
E = 320000
D = 128
K_HOPS = 2

NC = 2
NS = 16
NW = NC * NS
EPW = E // NW
CH = 125
NCHUNK = EPW // CH
NHALF = NCHUNK // 2
RPT = 624
RTAIL = N - NS * RPT
DW = D

def _sc_prop_body(src_hbm, dst_hbm, t_hbm, zeros_hbm, out_hbm,
                  acc_sh, srcidx, dstidx, rows0, rows1, sem0, sem1):
    c = lax.axis_index("c")
    s = lax.axis_index("s")
    pltpu.sync_copy(zeros_hbm.at[pl.ds(s * RPT, RPT)],
                    acc_sh.at[pl.ds(s * RPT, RPT)])

    @pl.when(s == NS - 1)
    def _():
        pltpu.sync_copy(zeros_hbm.at[pl.ds(NS * RPT, RTAIL)],
                        acc_sh.at[pl.ds(NS * RPT, RTAIL)])

    plsc.subcore_barrier()

    wid = c * NS + s
    for h in range(2):
        pltpu.sync_copy(src_hbm.at[wid].at[h], srcidx)
        pltpu.sync_copy(dst_hbm.at[wid].at[h], dstidx)
        pltpu.async_copy(t_hbm.at[srcidx.at[0]], rows0, sem0)

        def chunk(j, carry):
            i0 = 2 * j
            pltpu.make_async_copy(t_hbm.at[srcidx.at[i0]], rows0, sem0).wait()
            pltpu.async_copy(t_hbm.at[srcidx.at[i0 + 1]], rows1, sem1)
            pltpu.sync_copy(rows0, acc_sh.at[dstidx.at[i0]], add=True)
            pltpu.make_async_copy(t_hbm.at[srcidx.at[i0 + 1]], rows1,
                                  sem1).wait()

            @pl.when(j < NHALF // 2 - 1)
            def _():
                pltpu.async_copy(t_hbm.at[srcidx.at[i0 + 2]], rows0, sem0)

            pltpu.sync_copy(rows1, acc_sh.at[dstidx.at[i0 + 1]], add=True)
            return carry

        lax.fori_loop(0, NHALF // 2, chunk, 0)
    plsc.subcore_barrier()
    pltpu.sync_copy(acc_sh.at[pl.ds(s * RPT, RPT)],
                    out_hbm.at[c].at[pl.ds(s * RPT, RPT)])

    @pl.when(s == NS - 1)
    def _():
        pltpu.sync_copy(acc_sh.at[pl.ds(NS * RPT, RTAIL)],
                        out_hbm.at[c].at[pl.ds(NS * RPT, RTAIL)])


@functools.cache
def _get_sc_prop():
    mesh = plsc.VectorSubcoreMesh(core_axis_name="c", subcore_axis_name="s")
    return pl.kernel(
        _sc_prop_body,
        out_type=jax.ShapeDtypeStruct((NC, N, D), jnp.float32),
        mesh=mesh,
        scratch_types=[
            pltpu.VMEM_SHARED((N, D), jnp.float32),
            pltpu.VMEM((NHALF, CH), jnp.int32),
            pltpu.VMEM((NHALF, CH), jnp.int32),
            pltpu.VMEM((CH, D), jnp.float32),
            pltpu.VMEM((CH, D), jnp.float32),
            pltpu.SemaphoreType.DMA,
            pltpu.SemaphoreType.DMA,
        ],
    )


def _sc_prop(src, dst, t, zeros):
    return _get_sc_prop()(src, dst, t, zeros)


def _sc_deg_body(dst_hbm, ones_hbm, zeros_hbm, out_hbm, acc_sh, dstidx, ones_v):
    c = lax.axis_index("c")
    s = lax.axis_index("s")
    pltpu.sync_copy(zeros_hbm.at[pl.ds(s * RPT, RPT)],
                    acc_sh.at[pl.ds(s * RPT, RPT)])

    @pl.when(s == NS - 1)
    def _():
        pltpu.sync_copy(zeros_hbm.at[pl.ds(NS * RPT, RTAIL)],
                        acc_sh.at[pl.ds(NS * RPT, RTAIL)])

    plsc.subcore_barrier()
    wid = c * NS + s
    pltpu.sync_copy(ones_hbm, ones_v)
    for h in range(2):
        pltpu.sync_copy(dst_hbm.at[wid].at[h], dstidx)

        def chunk(i, carry):
            pltpu.sync_copy(ones_v, acc_sh.at[dstidx.at[i]], add=True)
            return carry

        lax.fori_loop(0, NHALF, chunk, 0)
    plsc.subcore_barrier()
    pltpu.sync_copy(acc_sh.at[pl.ds(s * RPT, RPT)],
                    out_hbm.at[c].at[pl.ds(s * RPT, RPT)])

    @pl.when(s == NS - 1)
    def _():
        pltpu.sync_copy(acc_sh.at[pl.ds(NS * RPT, RTAIL)],
                        out_hbm.at[c].at[pl.ds(NS * RPT, RTAIL)])


@functools.cache
def _get_sc_deg():
    mesh = plsc.VectorSubcoreMesh(core_axis_name="c", subcore_axis_name="s")
    return pl.kernel(
        _sc_deg_body,
        out_type=jax.ShapeDtypeStruct((NC, N, DW), jnp.float32),
        mesh=mesh,
        scratch_types=[
            pltpu.VMEM_SHARED((N, DW), jnp.float32),
            pltpu.VMEM((NHALF, CH), jnp.int32),
            pltpu.VMEM((CH, DW), jnp.float32),
        ],
    )


_BLK = 1000
_NBLK = N // _BLK


def _prep_body(x_ref, degp_ref, t0_ref, normb_ref):
    deg = degp_ref[0, :, :1] + degp_ref[1, :, :1]
    norm = jnp.broadcast_to(lax.rsqrt(jnp.maximum(deg, 1.0)), (_BLK, D))
    normb_ref[...] = norm
    t0_ref[...] = norm * jnp.nan_to_num(x_ref[...])


def _tc_prep(x, degp):
    return pl.pallas_call(
        _prep_body,
        grid=(_NBLK,),
        in_specs=[
            pl.BlockSpec((_BLK, D), lambda i: (i, 0)),
            pl.BlockSpec((NC, _BLK, DW), lambda i: (0, i, 0)),
        ],
        out_specs=[
            pl.BlockSpec((_BLK, D), lambda i: (i, 0)),
            pl.BlockSpec((_BLK, D), lambda i: (i, 0)),
        ],
        out_shape=[
            jax.ShapeDtypeStruct((N, D), jnp.float32),
            jax.ShapeDtypeStruct((N, D), jnp.float32),
        ],
    )(x, degp)


def _mid_body(sp_ref, normb_ref, t_ref):
    nb = normb_ref[...]
    t_ref[...] = nb * nb * (sp_ref[0] + sp_ref[1])


def _tc_mid(sp, normb):
    return pl.pallas_call(
        _mid_body,
        grid=(_NBLK,),
        in_specs=[
            pl.BlockSpec((NC, _BLK, D), lambda i: (0, i, 0)),
            pl.BlockSpec((_BLK, D), lambda i: (i, 0)),
        ],
        out_specs=pl.BlockSpec((_BLK, D), lambda i: (i, 0)),
        out_shape=jax.ShapeDtypeStruct((N, D), jnp.float32),
    )(sp, normb)


def _layer_body(feat_ref, s1_ref, s2_ref, normb_ref, w_ref, b_ref,
                h_ref, t_ref):
    nb = normb_ref[...]
    f = jnp.nan_to_num(feat_ref[...])
    h1 = nb * (s1_ref[0] + s1_ref[1])
    h2 = nb * (s2_ref[0] + s2_ref[1])
    cat = jnp.concatenate([f, h1, h2], axis=1)
    out = jnp.dot(cat, w_ref[...], preferred_element_type=jnp.float32)
    out = jnp.maximum(out + b_ref[...], 0.0)
    h_ref[...] = out
    t_ref[...] = nb * out


def _tc_layer(feat, s1p, s2p, normb, W, b):
    return pl.pallas_call(
        _layer_body,
        grid=(_NBLK,),
        in_specs=[
            pl.BlockSpec((_BLK, D), lambda i: (i, 0)),
            pl.BlockSpec((NC, _BLK, D), lambda i: (0, i, 0)),
            pl.BlockSpec((NC, _BLK, D), lambda i: (0, i, 0)),
            pl.BlockSpec((_BLK, D), lambda i: (i, 0)),
            pl.BlockSpec((3 * D, D), lambda i: (0, 0)),
            pl.BlockSpec((1, D), lambda i: (0, 0)),
        ],
        out_specs=[
            pl.BlockSpec((_BLK, D), lambda i: (i, 0)),
            pl.BlockSpec((_BLK, D), lambda i: (i, 0)),
        ],
        out_shape=[
            jax.ShapeDtypeStruct((N, D), jnp.float32),
            jax.ShapeDtypeStruct((N, D), jnp.float32),
        ],
    )(feat, s1p, s2p, normb, W, b)


def _layer3_body(feat_ref, s1_ref, s2_ref, normb_ref, w_ref, b_ref,
                 wc_ref, bc_ref, out_ref, acc_ref):
    i = pl.program_id(0)
    nb = normb_ref[...]
    f = jnp.nan_to_num(feat_ref[...])
    h1 = nb * (s1_ref[0] + s1_ref[1])
    h2 = nb * (s2_ref[0] + s2_ref[1])
    cat = jnp.concatenate([f, h1, h2], axis=1)
    out = jnp.dot(cat, w_ref[...], preferred_element_type=jnp.float32)
    out = jnp.maximum(out + b_ref[...], 0.0)

    @pl.when(i == 0)
    def _():
        acc_ref[...] = jnp.zeros_like(acc_ref)

    acc_ref[...] += jnp.sum(out, axis=0, keepdims=True)

    @pl.when(i == _NBLK - 1)
    def _():
        hg = acc_ref[...] * (1.0 / N)
        out_ref[...] = jnp.dot(hg, wc_ref[...],
                               preferred_element_type=jnp.float32) + bc_ref[...]


def _tc_layer3(feat, s1p, s2p, normb, W, b, Wc, bc):
    return pl.pallas_call(
        _layer3_body,
        grid=(_NBLK,),
        in_specs=[
            pl.BlockSpec((_BLK, D), lambda i: (i, 0)),
            pl.BlockSpec((NC, _BLK, D), lambda i: (0, i, 0)),
            pl.BlockSpec((NC, _BLK, D), lambda i: (0, i, 0)),
            pl.BlockSpec((_BLK, D), lambda i: (i, 0)),
            pl.BlockSpec((3 * D, D), lambda i: (0, 0)),
            pl.BlockSpec((1, D), lambda i: (0, 0)),
            pl.BlockSpec((D, 1), lambda i: (0, 0)),
            pl.BlockSpec((1, 1), lambda i: (0, 0)),
        ],
        out_specs=pl.BlockSpec((1, 1), lambda i: (0, 0)),
        out_shape=jax.ShapeDtypeStruct((1, 1), jnp.float32),
        scratch_shapes=[pltpu.VMEM((1, D), jnp.float32)],
    )(feat, s1p, s2p, normb, W, b, Wc, bc)


def kernel(x, edge_index, W1, b1, W2, b2, W3, b3, Wc, bc):
    src = edge_index[0].reshape(NW, 2, NHALF, CH)
    dst = edge_index[1].reshape(NW, 2, NHALF, CH)
    zeros = jnp.zeros((N, D), jnp.float32)
    ones_chunk = jnp.ones((CH, DW), jnp.float32)
    b1r = b1.reshape(1, D)
    b2r = b2.reshape(1, D)
    b3r = b3.reshape(1, D)
    bcr = bc.reshape(1, 1)

    degp = _get_sc_deg()(dst, ones_chunk, zeros)
    t, normb = _tc_prep(x, degp)

    feat = x
    for W, b in ((W1, b1r), (W2, b2r)):
        s1p = _sc_prop(src, dst, t, zeros)
        t1 = _tc_mid(s1p, normb)
        s2p = _sc_prop(src, dst, t1, zeros)
        feat, t = _tc_layer(feat, s1p, s2p, normb, W, b)

    s1p = _sc_prop(src, dst, t, zeros)
    t1 = _tc_mid(s1p, normb)
    s2p = _sc_prop(src, dst, t1, zeros)
    return _tc_layer3(feat, s1p, s2p, normb, W3, b3r, Wc, bcr)
